# baseline (speedup 1.0000x reference)
import jax
import jax.numpy as jnp
from jax import lax
from jax.experimental import pallas as pl
from jax.experimental.pallas import tpu as pltpu
from jax.experimental.pallas import tpu_sc as plsc

POOL = 1000
POOL_PAD = 1024
TOPK = 4
PROMPT_LEN = 8
HIDDEN = 1024
QDIM = 2 * HIDDEN
BATCH = 1024

QBLK = 256
NSTEPS = BATCH // QBLK

NIDX = BATCH * TOPK * PROMPT_LEN
NWORKERS = 32
PER_W = NIDX // NWORKERS
CHUNK = 16
NBUF = 4
NCHUNK = PER_W // CHUNK


def _topk_body(q_ref, k_ref, sim_ref, idx_ref, loss_ref):
    step = pl.program_id(0)
    q = q_ref[...]
    km = k_ref[...]
    qn = jnp.sqrt(jnp.sum(q * q, axis=1, keepdims=True))
    kn_col = jnp.sqrt(jnp.sum(km * km, axis=1, keepdims=True))
    kn = jnp.sqrt(jnp.sum(km * km, axis=1))[None, :]

    dots = jax.lax.dot_general(
        q, km, dimension_numbers=(((1,), (1,)), ((), ())),
        preferred_element_type=jnp.float32)
    sim = dots / jnp.maximum(qn * kn, 1e-8)

    qnorm = q / jnp.maximum(qn, 1e-12)
    knorm = km / jnp.maximum(kn_col, 1e-12)
    sim2 = jax.lax.dot_general(
        qnorm, knorm, dimension_numbers=(((1,), (1,)), ((), ())),
        preferred_element_type=jnp.float32)
    part = jnp.full((1, 1), -jnp.sum(sim2) / (BATCH * POOL), jnp.float32)

    @pl.when(step == 0)
    def _():
        loss_ref[...] = jnp.zeros((1, 1), jnp.float32)
    loss_ref[...] += part

    col = jax.lax.broadcasted_iota(jnp.int32, (QBLK, POOL_PAD), 1)
    iota8 = jax.lax.broadcasted_iota(jnp.int32, (QBLK, PROMPT_LEN), 1)
    m = jnp.where(col < POOL, sim, -2.0)
    for k in range(TOPK):
        mx = jnp.max(m, axis=1, keepdims=True)
        amax = jnp.min(jnp.where(m == mx, col, 2**30), axis=1, keepdims=True)
        sim_ref[:, pl.ds(k, 1)] = mx
        idx_ref[:, pl.ds(k * PROMPT_LEN, PROMPT_LEN)] = amax * PROMPT_LEN + iota8
        m = jnp.where(col == amax, -3.0, m)


def _topk_call(querys, pk_pad):
    b = querys.shape[0]
    return pl.pallas_call(
        _topk_body,
        grid=(b // QBLK,),
        in_specs=[
            pl.BlockSpec((QBLK, QDIM), lambda i: (i, 0)),
            pl.BlockSpec((POOL_PAD, QDIM), lambda i: (0, 0)),
        ],
        out_specs=[
            pl.BlockSpec((QBLK, TOPK), lambda i: (i, 0)),
            pl.BlockSpec((QBLK, TOPK * PROMPT_LEN), lambda i: (i, 0)),
            pl.BlockSpec((1, 1), lambda i: (0, 0)),
        ],
        out_shape=[
            jax.ShapeDtypeStruct((b, TOPK), jnp.float32),
            jax.ShapeDtypeStruct((b, TOPK * PROMPT_LEN), jnp.int32),
            jax.ShapeDtypeStruct((1, 1), jnp.float32),
        ],
    )(querys, pk_pad)


def _gather_loop(flat_hbm, idx_hbm, out_hbm, idx_v, bufs_sems, n, row_off):
    per_w = n // NWORKERS
    nchunk = per_w // CHUNK
    bufs = bufs_sems[:NBUF]
    gsems = bufs_sems[NBUF:2 * NBUF]
    osems = bufs_sems[2 * NBUF:]
    wid = lax.axis_index("s") * 2 + lax.axis_index("c")
    base = wid * per_w
    obase = row_off + base
    pltpu.sync_copy(idx_hbm.at[pl.ds(base, per_w)], idx_v)

    def start_gather(c, p):
        pltpu.make_async_copy(
            flat_hbm.at[idx_v.at[pl.ds(c * CHUNK, CHUNK)]],
            bufs[p], gsems[p]).start()

    def wait_gather(p):
        pltpu.make_async_copy(
            flat_hbm.at[idx_v.at[pl.ds(0, CHUNK)]],
            bufs[p], gsems[p]).wait()

    def start_wb(c, p):
        pltpu.make_async_copy(
            bufs[p], out_hbm.at[pl.ds(obase + c * CHUNK, CHUNK)],
            osems[p]).start()

    def wait_wb(p):
        pltpu.make_async_copy(
            bufs[p], out_hbm.at[pl.ds(obase, CHUNK)], osems[p]).wait()

    for p in range(NBUF):
        start_gather(p, p)

    @pl.loop(0, nchunk, step=NBUF)
    def _(c):
        for p in range(NBUF):
            ch = c + p
            wait_gather(p)
            start_wb(ch, p)
            nxt = ch + NBUF

            @pl.when(nxt < nchunk)
            def _():
                wait_wb(p)
                start_gather(nxt, p)

    for p in range(NBUF):
        wait_wb(p)


def _sc_scratch(per_w):
    return ([pltpu.VMEM((per_w,), jnp.int32)]
            + [pltpu.VMEM((CHUNK, HIDDEN), jnp.float32)] * NBUF
            + [pltpu.SemaphoreType.DMA] * NBUF
            + [pltpu.SemaphoreType.DMA] * NBUF)


def _sc_gather_alloc(flat, idx, out_rows):
    n = idx.shape[0]
    mesh = plsc.VectorSubcoreMesh(core_axis_name="c", subcore_axis_name="s")

    @pl.kernel(
        out_type=jax.ShapeDtypeStruct((out_rows, HIDDEN), jnp.float32),
        mesh=mesh,
        scratch_types=_sc_scratch(n // NWORKERS),
    )
    def gk(flat_hbm, idx_hbm, out_hbm, idx_v, *bufs_sems):
        _gather_loop(flat_hbm, idx_hbm, out_hbm, idx_v, bufs_sems, n, 0)

    return gk(flat, idx)


def _sc_gather_into(flat, idx, out_ref, row_off):
    n = idx.shape[0]
    mesh = plsc.VectorSubcoreMesh(core_axis_name="c", subcore_axis_name="s")

    @pl.kernel(
        out_type=(),
        mesh=mesh,
        scratch_types=_sc_scratch(n // NWORKERS),
    )
    def gk(flat_hbm, idx_hbm, out_hbm, idx_v, *bufs_sems):
        _gather_loop(flat_hbm, idx_hbm, out_hbm, idx_v, bufs_sems, n, row_off)

    gk(flat, idx, out_ref)


def kernel(querys, prompts_key, prompts):
    pk_pad = jnp.pad(prompts_key, ((0, POOL_PAD - POOL), (0, 0)))
    flat = prompts.reshape(POOL * PROMPT_LEN, HIDDEN)
    half = BATCH // 2
    nh = NIDX // 2
    sim0, idx0, loss0 = _topk_call(querys[:half], pk_pad)
    g0 = _sc_gather_alloc(flat, idx0.reshape(nh), out_rows=NIDX)
    sim1, idx1, loss1 = _topk_call(querys[half:], pk_pad)
    out_ref = jax.new_ref(g0)
    _sc_gather_into(flat, idx1.reshape(nh), out_ref, row_off=nh)
    selected = out_ref[...].reshape(BATCH, TOPK, PROMPT_LEN, HIDDEN)
    sim_topk = jnp.concatenate([sim0, sim1], axis=0)
    loss = (loss0 + loss1).reshape(())
    return selected, sim_topk, loss
